# Initial kernel scaffold; baseline (speedup 1.0000x reference)
import jax
import jax.numpy as jnp
from jax.experimental import pallas as pl


def kernel(x1, x2, edge_index1, edge_index2, edge_type1, edge_type2, Wemb1, bemb1, Wemb2, bemb2, We1, be1, We2, be2, Wih1, Whh1, bih1, bhh1, Wih2, Whh2, bih2, bhh2, gamma1, beta1, gamma2, beta2, Wf, bf):
    raise NotImplementedError("write your pallas kernel here")


# speedup vs baseline: 4.1984x; 4.1984x over previous
import functools

import jax
import jax.numpy as jnp
from jax.experimental import pallas as pl

N = 10000
E = 320000
D = 128
NETYPES = 4
NSTEPS = 6
CLASS_NUM = 2


def _leaky(x):
    return jax.nn.leaky_relu(x, 0.01)


def _branch(x, ei, et, Wemb, bemb, We, be, Wih, Whh, bih, bhh, g, b):
    src, dst = ei[0], ei[1]
    emb = x @ Wemb.T + bemb
    h = emb
    Wcat = jnp.transpose(We, (2, 0, 1)).reshape(D, NETYPES * D)
    bcat = be.reshape(NETYPES * D)
    gidx = src * NETYPES + et
    for _ in range(NSTEPS):
        M = (h @ Wcat + bcat).reshape(N * NETYPES, D)
        msg = M[gidx]
        a = jnp.zeros((N, D), h.dtype).at[dst].add(msg)
        gi = a @ Wih.T + bih
        gh = h @ Whh.T + bhh
        i_r, i_z, i_n = jnp.split(gi, 3, axis=1)
        h_r, h_z, h_n = jnp.split(gh, 3, axis=1)
        r = jax.nn.sigmoid(i_r + h_r)
        z = jax.nn.sigmoid(i_z + h_z)
        n_t = jnp.tanh(i_n + r * h_n)
        h = (1.0 - z) * n_t + z * h
    h = _leaky(h)
    hc = jnp.concatenate([h, emb], axis=-1)
    mu = jnp.mean(hc, axis=0)
    var = jnp.var(hc, axis=0)
    hn = (hc - mu) / jnp.sqrt(var + 1e-5) * g + b
    return jnp.mean(hn, axis=0, keepdims=True)


def _cls_body(feats_ref, wf_ref, bf_ref, out_ref):
    feats = feats_ref[...]
    logits = jax.lax.dot_general(
        feats, wf_ref[...], (((1,), (1,)), ((), ())),
        preferred_element_type=jnp.float32) + bf_ref[...]
    logits = _leaky(logits)
    m = jnp.max(logits, axis=-1, keepdims=True)
    e = jnp.exp(logits - m)
    out_ref[...] = e / jnp.sum(e, axis=-1, keepdims=True)


@jax.jit
def _classifier(feats, Wf, bf):
    return pl.pallas_call(
        _cls_body,
        out_shape=jax.ShapeDtypeStruct((1, CLASS_NUM), jnp.float32),
    )(feats, Wf, bf.reshape(1, CLASS_NUM))


def kernel(x1, x2, edge_index1, edge_index2, edge_type1, edge_type2,
           Wemb1, bemb1, Wemb2, bemb2, We1, be1, We2, be2,
           Wih1, Whh1, bih1, bhh1, Wih2, Whh2, bih2, bhh2,
           gamma1, beta1, gamma2, beta2, Wf, bf):
    m1 = _branch(x1, edge_index1, edge_type1, Wemb1, bemb1, We1, be1,
                 Wih1, Whh1, bih1, bhh1, gamma1, beta1)
    m2 = _branch(x2, edge_index2, edge_type2, Wemb2, bemb2, We2, be2,
                 Wih2, Whh2, bih2, bhh2, gamma2, beta2)
    feats = jnp.concatenate([m1, m2], axis=1)
    return _classifier(feats, Wf, bf)


# speedup vs baseline: 11.2009x; 2.6679x over previous
import functools

import jax
import jax.numpy as jnp
from jax import lax
from jax.experimental import pallas as pl
from jax.experimental.pallas import tpu as pltpu
from jax.experimental.pallas import tpu_sc as plsc

N = 10000
E = 320000
D = 128
NETYPES = 4
NSTEPS = 6
CLASS_NUM = 2

NCORES = 2
NSUB = 16
NW = NCORES * NSUB
CHUNK = 128
CPW = 79
EPAD = NW * CPW * CHUNK
NPAD = 10112
STRIPE = NPAD // NSUB

BN = 2000
GRID = N // BN

_leaky = functools.partial(jax.nn.leaky_relu, negative_slope=0.01)


def _edge_body(m_hbm, gidx_hbm, dst_hbm, zeros_hbm, out_hbm,
               gidx_v, dst_v, rows_v, acc, sem):
    c = lax.axis_index("c")
    s = lax.axis_index("s")
    w = c * NSUB + s
    pltpu.sync_copy(gidx_hbm.at[w], gidx_v)
    pltpu.sync_copy(dst_hbm.at[w], dst_v)
    pltpu.sync_copy(zeros_hbm.at[pl.ds(s * STRIPE, STRIPE)],
                    acc.at[pl.ds(s * STRIPE, STRIPE)])
    plsc.subcore_barrier()

    def chunk(j, carry):
        pltpu.async_copy(m_hbm.at[gidx_v.at[j]], rows_v, sem).wait()
        pltpu.sync_copy(rows_v, acc.at[dst_v.at[j]], add=True)
        return carry

    lax.fori_loop(0, CPW, chunk, 0)
    plsc.subcore_barrier()
    pltpu.sync_copy(acc.at[pl.ds(s * STRIPE, STRIPE)],
                    out_hbm.at[c, pl.ds(s * STRIPE, STRIPE)])


_edge_kernel = pl.kernel(
    _edge_body,
    out_type=jax.ShapeDtypeStruct((NCORES, NPAD, D), jnp.float32),
    mesh=plsc.VectorSubcoreMesh(core_axis_name="c", subcore_axis_name="s"),
    scratch_types=[
        pltpu.VMEM((CPW, CHUNK), jnp.int32),
        pltpu.VMEM((CPW, CHUNK), jnp.int32),
        pltpu.VMEM((CHUNK, D), jnp.float32),
        pltpu.VMEM_SHARED((NPAD, D), jnp.float32),
        pltpu.SemaphoreType.DMA,
    ],
)


def _init_body(x_ref, wemb_ref, bemb_ref, wcat_ref, bcat_ref, emb_ref, m_ref):
    e = lax.dot_general(x_ref[...], wemb_ref[...], (((1,), (1,)), ((), ())),
                        preferred_element_type=jnp.float32) + bemb_ref[...]
    emb_ref[...] = e
    m = lax.dot_general(e, wcat_ref[...], (((1,), (0,)), ((), ())),
                        preferred_element_type=jnp.float32) + bcat_ref[...]
    for i in range(NETYPES):
        m_ref[i] = m[:, i * D:(i + 1) * D]


def _init_call(x, wemb, bemb, wcat, bcat):
    return pl.pallas_call(
        _init_body,
        grid=(GRID,),
        in_specs=[
            pl.BlockSpec((BN, D), lambda i: (i, 0)),
            pl.BlockSpec((D, D), lambda i: (0, 0)),
            pl.BlockSpec((1, D), lambda i: (0, 0)),
            pl.BlockSpec((D, NETYPES * D), lambda i: (0, 0)),
            pl.BlockSpec((1, NETYPES * D), lambda i: (0, 0)),
        ],
        out_specs=[
            pl.BlockSpec((BN, D), lambda i: (i, 0)),
            pl.BlockSpec((NETYPES, BN, D), lambda i: (0, i, 0)),
        ],
        out_shape=[
            jax.ShapeDtypeStruct((N, D), jnp.float32),
            jax.ShapeDtypeStruct((NETYPES, N, D), jnp.float32),
        ],
    )(x, wemb, bemb, wcat, bcat)


def _step_body(p_ref, h_ref, wih_ref, whh_ref, bih_ref, bhh_ref,
               wcat_ref, bcat_ref, hout_ref, m_ref):
    a = p_ref[0] + p_ref[1]
    h = h_ref[...]
    gi = lax.dot_general(a, wih_ref[...], (((1,), (1,)), ((), ())),
                         preferred_element_type=jnp.float32) + bih_ref[...]
    gh = lax.dot_general(h, whh_ref[...], (((1,), (1,)), ((), ())),
                         preferred_element_type=jnp.float32) + bhh_ref[...]
    r = jax.nn.sigmoid(gi[:, :D] + gh[:, :D])
    z = jax.nn.sigmoid(gi[:, D:2 * D] + gh[:, D:2 * D])
    nt = jnp.tanh(gi[:, 2 * D:] + r * gh[:, 2 * D:])
    hn = (1.0 - z) * nt + z * h
    hout_ref[...] = hn
    m = lax.dot_general(hn, wcat_ref[...], (((1,), (0,)), ((), ())),
                        preferred_element_type=jnp.float32) + bcat_ref[...]
    for i in range(NETYPES):
        m_ref[i] = m[:, i * D:(i + 1) * D]


def _step_call(p, h, wih, whh, bih, bhh, wcat, bcat):
    return pl.pallas_call(
        _step_body,
        grid=(GRID,),
        in_specs=[
            pl.BlockSpec((NCORES, BN, D), lambda i: (0, i, 0)),
            pl.BlockSpec((BN, D), lambda i: (i, 0)),
            pl.BlockSpec((3 * D, D), lambda i: (0, 0)),
            pl.BlockSpec((3 * D, D), lambda i: (0, 0)),
            pl.BlockSpec((1, 3 * D), lambda i: (0, 0)),
            pl.BlockSpec((1, 3 * D), lambda i: (0, 0)),
            pl.BlockSpec((D, NETYPES * D), lambda i: (0, 0)),
            pl.BlockSpec((1, NETYPES * D), lambda i: (0, 0)),
        ],
        out_specs=[
            pl.BlockSpec((BN, D), lambda i: (i, 0)),
            pl.BlockSpec((NETYPES, BN, D), lambda i: (0, i, 0)),
        ],
        out_shape=[
            jax.ShapeDtypeStruct((N, D), jnp.float32),
            jax.ShapeDtypeStruct((NETYPES, N, D), jnp.float32),
        ],
    )(p, h, wih, whh, bih, bhh, wcat, bcat)


def _stats_body(h_ref, emb_ref, o_ref):
    i = pl.program_id(0)
    hc = jnp.concatenate([_leaky(h_ref[...]), emb_ref[...]], axis=1)
    st = jnp.concatenate([jnp.sum(hc, axis=0, keepdims=True),
                          jnp.sum(hc * hc, axis=0, keepdims=True)], axis=0)

    @pl.when(i == 0)
    def _():
        o_ref[...] = st

    @pl.when(i != 0)
    def _():
        o_ref[...] += st


def _norm_body(h_ref, emb_ref, st_ref, g_ref, b_ref, o_ref):
    i = pl.program_id(0)
    hc = jnp.concatenate([_leaky(h_ref[...]), emb_ref[...]], axis=1)
    mu = st_ref[0:1, :] * (1.0 / N)
    var = st_ref[1:2, :] * (1.0 / N) - mu * mu
    rstd = lax.rsqrt(var + 1e-5)
    contrib = jnp.sum((hc - mu) * rstd * g_ref[...] + b_ref[...],
                      axis=0, keepdims=True)

    @pl.when(i == 0)
    def _():
        o_ref[...] = contrib

    @pl.when(i != 0)
    def _():
        o_ref[...] += contrib

    @pl.when(i == GRID - 1)
    def _():
        o_ref[...] *= (1.0 / N)


def _readout(h, emb, g, b):
    stats = pl.pallas_call(
        _stats_body,
        grid=(GRID,),
        in_specs=[pl.BlockSpec((BN, D), lambda i: (i, 0)),
                  pl.BlockSpec((BN, D), lambda i: (i, 0))],
        out_specs=pl.BlockSpec((2, 2 * D), lambda i: (0, 0)),
        out_shape=jax.ShapeDtypeStruct((2, 2 * D), jnp.float32),
    )(h, emb)
    return pl.pallas_call(
        _norm_body,
        grid=(GRID,),
        in_specs=[pl.BlockSpec((BN, D), lambda i: (i, 0)),
                  pl.BlockSpec((BN, D), lambda i: (i, 0)),
                  pl.BlockSpec((2, 2 * D), lambda i: (0, 0)),
                  pl.BlockSpec((1, 2 * D), lambda i: (0, 0)),
                  pl.BlockSpec((1, 2 * D), lambda i: (0, 0))],
        out_specs=pl.BlockSpec((1, 2 * D), lambda i: (0, 0)),
        out_shape=jax.ShapeDtypeStruct((1, 2 * D), jnp.float32),
    )(h, emb, stats, g.reshape(1, 2 * D), b.reshape(1, 2 * D))


def _cls_body(feats_ref, wf_ref, bf_ref, out_ref):
    logits = lax.dot_general(feats_ref[...], wf_ref[...],
                             (((1,), (1,)), ((), ())),
                             preferred_element_type=jnp.float32) + bf_ref[...]
    logits = _leaky(logits)
    m = jnp.max(logits, axis=-1, keepdims=True)
    e = jnp.exp(logits - m)
    out_ref[...] = e / jnp.sum(e, axis=-1, keepdims=True)


def _classifier(feats, wf, bf):
    return pl.pallas_call(
        _cls_body,
        out_shape=jax.ShapeDtypeStruct((1, CLASS_NUM), jnp.float32),
    )(feats, wf, bf.reshape(1, CLASS_NUM))


def _prep_edges(ei, et):
    src, dst = ei[0], ei[1]
    gidx = et * N + src
    pad = EPAD - E
    gidx = jnp.concatenate([gidx, jnp.zeros((pad,), jnp.int32)])
    dstp = jnp.concatenate([dst, jnp.full((pad,), N, jnp.int32)])
    return gidx.reshape(NW, CPW, CHUNK), dstp.reshape(NW, CPW, CHUNK)


def _prep_w(we, be):
    wcat = jnp.transpose(we, (2, 0, 1)).reshape(D, NETYPES * D)
    bcat = be.reshape(1, NETYPES * D)
    return wcat, bcat


def kernel(x1, x2, edge_index1, edge_index2, edge_type1, edge_type2,
           Wemb1, bemb1, Wemb2, bemb2, We1, be1, We2, be2,
           Wih1, Whh1, bih1, bhh1, Wih2, Whh2, bih2, bhh2,
           gamma1, beta1, gamma2, beta2, Wf, bf):
    zeros = jnp.zeros((NPAD, D), jnp.float32)
    g1, d1 = _prep_edges(edge_index1, edge_type1)
    g2, d2 = _prep_edges(edge_index2, edge_type2)
    wc1, bc1 = _prep_w(We1, be1)
    wc2, bc2 = _prep_w(We2, be2)
    emb1, M1 = _init_call(x1, Wemb1, bemb1.reshape(1, D), wc1, bc1)
    emb2, M2 = _init_call(x2, Wemb2, bemb2.reshape(1, D), wc2, bc2)
    h1, h2 = emb1, emb2
    bih1r, bhh1r = bih1.reshape(1, 3 * D), bhh1.reshape(1, 3 * D)
    bih2r, bhh2r = bih2.reshape(1, 3 * D), bhh2.reshape(1, 3 * D)
    for _ in range(NSTEPS):
        p1 = _edge_kernel(M1.reshape(NETYPES * N, D), g1, d1, zeros)
        p2 = _edge_kernel(M2.reshape(NETYPES * N, D), g2, d2, zeros)
        h1, M1 = _step_call(p1, h1, Wih1, Whh1, bih1r, bhh1r, wc1, bc1)
        h2, M2 = _step_call(p2, h2, Wih2, Whh2, bih2r, bhh2r, wc2, bc2)
    m1 = _readout(h1, emb1, gamma1, beta1)
    m2 = _readout(h2, emb2, gamma2, beta2)
    feats = jnp.concatenate([m1, m2], axis=1)
    return _classifier(feats, Wf, bf)
